# Initial kernel scaffold; baseline (speedup 1.0000x reference)
import jax
import jax.numpy as jnp
from jax.experimental import pallas as pl


def kernel(atom_feat, bond_feat, pair_idx, kernel, bias):
    raise NotImplementedError("write your pallas kernel here")


# speedup vs baseline: 1.8916x; 1.8916x over previous
import functools

import jax
import jax.numpy as jnp
from jax import lax
from jax.experimental import pallas as pl
from jax.experimental.pallas import tpu as pltpu
from jax.experimental.pallas import tpu_sc as plsc

E = 71680
NN = 4480
AD = 32
BD = 16
NC = 2
NS = 16
NW = NC * NS
EPW = E // NW
CH = 112
NCH = EPW // CH
RPT = NN // NS
BE = 1024
K_AUG = BD * AD + AD


def _sc_mesh():
    return plsc.VectorSubcoreMesh(core_axis_name="c", subcore_axis_name="s")


_SC_PARAMS = pltpu.CompilerParams(use_tc_tiling_on_sc=False)


def _gather(atom_feat, dst_idx):
    @functools.partial(
        pl.kernel,
        out_type=jax.ShapeDtypeStruct((E, AD), jnp.float32),
        mesh=_sc_mesh(),
        scratch_types=[
            pltpu.VMEM((NCH, CH), jnp.int32),
            pltpu.VMEM((EPW, AD), jnp.float32),
            pltpu.SemaphoreType.DMA,
        ],
        compiler_params=_SC_PARAMS,
    )
    def gather_kernel(table_hbm, idx_hbm, out_hbm, idx_v, rows_v, sem):
        c = lax.axis_index("c")
        s = lax.axis_index("s")
        wid = s * NC + c
        pltpu.sync_copy(idx_hbm.at[wid], idx_v)
        copies = []
        for j in range(NCH):
            copies.append(
                pltpu.async_copy(
                    table_hbm.at[idx_v.at[j]],
                    rows_v.at[pl.ds(j * CH, CH)],
                    sem,
                )
            )
        for cp in copies:
            cp.wait()
        pltpu.sync_copy(rows_v, out_hbm.at[pl.ds(wid * EPW, EPW)])

    return gather_kernel(atom_feat, dst_idx)


def _dense_body(b_ref, n_ref, w_ref, o_ref):
    bf = b_ref[...]
    nb = n_ref[...]
    parts = [bf[:, k:k + 1] * nb for k in range(BD)]
    parts.append(nb)
    z = jnp.concatenate(parts, axis=1)
    o_ref[...] = jnp.dot(
        z, w_ref[...],
        preferred_element_type=jnp.float32,
        precision=lax.Precision.HIGHEST,
    )


def _dense(bond_feat, neighbor, w2_aug):
    return pl.pallas_call(
        _dense_body,
        grid=(E // BE,),
        in_specs=[
            pl.BlockSpec((BE, BD), lambda i: (i, 0)),
            pl.BlockSpec((BE, AD), lambda i: (i, 0)),
            pl.BlockSpec((K_AUG, AD), lambda i: (0, 0)),
        ],
        out_specs=pl.BlockSpec((BE, AD), lambda i: (i, 0)),
        out_shape=jax.ShapeDtypeStruct((E, AD), jnp.float32),
    )(bond_feat, neighbor, w2_aug)


def _scatter(msg, src_idx, zeros):
    @functools.partial(
        pl.kernel,
        out_type=jax.ShapeDtypeStruct((NC, NN, AD), jnp.float32),
        mesh=_sc_mesh(),
        scratch_types=[
            pltpu.VMEM((NCH, CH), jnp.int32),
            pltpu.VMEM((EPW, AD), jnp.float32),
            pltpu.VMEM_SHARED((NN, AD), jnp.float32),
            pltpu.SemaphoreType.DMA,
        ],
        compiler_params=_SC_PARAMS,
    )
    def scatter_kernel(msg_hbm, idx_hbm, zero_hbm, out_hbm, idx_v, msg_v, acc, sem):
        c = lax.axis_index("c")
        s = lax.axis_index("s")
        wid = s * NC + c
        pltpu.sync_copy(
            zero_hbm.at[pl.ds(s * RPT, RPT)], acc.at[pl.ds(s * RPT, RPT)]
        )
        pltpu.sync_copy(idx_hbm.at[wid], idx_v)
        pltpu.sync_copy(msg_hbm.at[pl.ds(wid * EPW, EPW)], msg_v)
        plsc.subcore_barrier()
        for j in range(NCH):
            pltpu.sync_copy(
                msg_v.at[pl.ds(j * CH, CH)], acc.at[idx_v.at[j]], add=True
            )
        plsc.subcore_barrier()
        pltpu.sync_copy(
            acc.at[pl.ds(s * RPT, RPT)], out_hbm.at[c, pl.ds(s * RPT, RPT)]
        )

    return scatter_kernel(msg, src_idx, zeros)


def _combine_body(p_ref, o_ref):
    o_ref[...] = p_ref[0] + p_ref[1]


def _combine(partials):
    return pl.pallas_call(
        _combine_body,
        out_shape=jax.ShapeDtypeStruct((NN, AD), jnp.float32),
    )(partials)


def kernel(atom_feat, bond_feat, pair_idx, kernel, bias):
    w2 = kernel.reshape(BD, AD, AD).transpose(0, 2, 1).reshape(BD * AD, AD)
    w2_aug = jnp.concatenate([w2, bias.reshape(AD, AD).T], axis=0)
    idx = pair_idx.astype(jnp.int32)
    src = idx[:, 0].reshape(NW, NCH, CH)
    dst = idx[:, 1].reshape(NW, NCH, CH)

    neighbor = _gather(atom_feat, dst)
    msg = _dense(bond_feat, neighbor, w2_aug)
    partials = _scatter(msg, src, jnp.zeros((NN, AD), jnp.float32))
    return _combine(partials)


# speedup vs baseline: 2.5671x; 1.3571x over previous
import functools

import jax
import jax.numpy as jnp
from jax import lax
from jax.experimental import pallas as pl
from jax.experimental.pallas import tpu as pltpu
from jax.experimental.pallas import tpu_sc as plsc

E = 71680
NN = 4480
AD = 32
BD = 16
NC = 2
NS = 16
NW = NC * NS
EPW = E // NW
CH = 112
NCH = EPW // CH
RPT = NN // NS
BE = 2048
K_AUG = BD * AD + AD


def _sc_mesh():
    return plsc.VectorSubcoreMesh(core_axis_name="c", subcore_axis_name="s")


_SC_PARAMS = pltpu.CompilerParams(use_tc_tiling_on_sc=False)


def _gather(atom_feat, dst_idx):
    @functools.partial(
        pl.kernel,
        out_type=jax.ShapeDtypeStruct((E, AD), jnp.float32),
        mesh=_sc_mesh(),
        scratch_types=[
            pltpu.VMEM((NCH, CH), jnp.int32),
            pltpu.VMEM((EPW, AD), jnp.float32),
            pltpu.SemaphoreType.DMA,
        ],
        compiler_params=_SC_PARAMS,
    )
    def gather_kernel(table_hbm, idx_hbm, out_hbm, idx_v, rows_v, sem):
        c = lax.axis_index("c")
        s = lax.axis_index("s")
        wid = s * NC + c
        pltpu.sync_copy(idx_hbm.at[wid], idx_v)
        copies = []
        for j in range(NCH):
            copies.append(
                pltpu.async_copy(
                    table_hbm.at[idx_v.at[j]],
                    rows_v.at[pl.ds(j * CH, CH)],
                    sem,
                )
            )
        for cp in copies:
            cp.wait()
        pltpu.sync_copy(rows_v, out_hbm.at[pl.ds(wid * EPW, EPW)])

    return gather_kernel(atom_feat, dst_idx)


def _dense_body(b_ref, n_ref, w_ref, o_ref):
    bf = b_ref[...]
    nb = n_ref[...]
    parts = [bf[:, k:k + 1] * nb for k in range(BD)]
    parts.append(nb)
    z = jnp.concatenate(parts, axis=1)
    o_ref[...] = jnp.dot(z, w_ref[...], preferred_element_type=jnp.float32)


def _dense(bond_feat, neighbor, w2_aug):
    return pl.pallas_call(
        _dense_body,
        grid=(E // BE,),
        in_specs=[
            pl.BlockSpec((BE, BD), lambda i: (i, 0)),
            pl.BlockSpec((BE, AD), lambda i: (i, 0)),
            pl.BlockSpec((K_AUG, AD), lambda i: (0, 0)),
        ],
        out_specs=pl.BlockSpec((BE, AD), lambda i: (i, 0)),
        out_shape=jax.ShapeDtypeStruct((E, AD), jnp.float32),
    )(bond_feat, neighbor, w2_aug)


def _scatter(msg, src_idx, zeros):
    @functools.partial(
        pl.kernel,
        out_type=jax.ShapeDtypeStruct((NC, NN, AD), jnp.float32),
        mesh=_sc_mesh(),
        scratch_types=[
            pltpu.VMEM((NCH, CH), jnp.int32),
            pltpu.VMEM((EPW, AD), jnp.float32),
            pltpu.VMEM_SHARED((NN, AD), jnp.float32),
            pltpu.SemaphoreType.DMA,
        ],
        compiler_params=_SC_PARAMS,
    )
    def scatter_kernel(msg_hbm, idx_hbm, zero_hbm, out_hbm, idx_v, msg_v, acc, sem):
        c = lax.axis_index("c")
        s = lax.axis_index("s")
        wid = s * NC + c
        pltpu.sync_copy(
            zero_hbm.at[pl.ds(s * RPT, RPT)], acc.at[pl.ds(s * RPT, RPT)]
        )
        pltpu.sync_copy(idx_hbm.at[wid], idx_v)
        pltpu.sync_copy(msg_hbm.at[pl.ds(wid * EPW, EPW)], msg_v)
        plsc.subcore_barrier()
        for j in range(NCH):
            pltpu.sync_copy(
                msg_v.at[pl.ds(j * CH, CH)], acc.at[idx_v.at[j]], add=True
            )
        plsc.subcore_barrier()
        pltpu.sync_copy(
            acc.at[pl.ds(s * RPT, RPT)], out_hbm.at[c, pl.ds(s * RPT, RPT)]
        )

    return scatter_kernel(msg, src_idx, zeros)


def _combine_body(p_ref, o_ref):
    o_ref[...] = p_ref[0] + p_ref[1]


def _combine(partials):
    return pl.pallas_call(
        _combine_body,
        out_shape=jax.ShapeDtypeStruct((NN, AD), jnp.float32),
    )(partials)


def kernel(atom_feat, bond_feat, pair_idx, kernel, bias):
    w2 = kernel.reshape(BD, AD, AD).transpose(0, 2, 1).reshape(BD * AD, AD)
    w2_aug = jnp.concatenate([w2, bias.reshape(AD, AD).T], axis=0)
    idx = pair_idx.astype(jnp.int32)
    src = idx[:, 0].reshape(NW, NCH, CH)
    dst = idx[:, 1].reshape(NW, NCH, CH)

    neighbor = _gather(atom_feat, dst)
    msg = _dense(bond_feat, neighbor, w2_aug)
    partials = _scatter(msg, src, jnp.zeros((NN, AD), jnp.float32))
    return _combine(partials)


# speedup vs baseline: 5.4453x; 2.1211x over previous
import functools

import jax
import jax.numpy as jnp
from jax import lax
from jax.experimental import pallas as pl
from jax.experimental.pallas import tpu as pltpu
from jax.experimental.pallas import tpu_sc as plsc

E = 71680
NN = 4480
AD = 32
BD = 16
NC = 2
NS = 16
NW = NC * NS
EPW = E // NW
CH = 112
NCH = EPW // CH
RPT = NN // NS
BE = 2048
K_AUG = BD * AD + AD


def _sc_mesh():
    return plsc.VectorSubcoreMesh(core_axis_name="c", subcore_axis_name="s")


_SC_PARAMS = pltpu.CompilerParams(use_tc_tiling_on_sc=False)


def _gather(atom_feat, dst_idx):
    @functools.partial(
        pl.kernel,
        out_type=jax.ShapeDtypeStruct((E, AD), jnp.float32),
        mesh=_sc_mesh(),
        scratch_types=[
            pltpu.VMEM((NCH, CH), jnp.int32),
            pltpu.VMEM((EPW, AD), jnp.float32),
            pltpu.SemaphoreType.DMA,
        ],
        compiler_params=_SC_PARAMS,
    )
    def gather_kernel(table_hbm, idx_hbm, out_hbm, idx_v, rows_v, sem):
        c = lax.axis_index("c")
        s = lax.axis_index("s")
        wid = s * NC + c
        pltpu.sync_copy(idx_hbm.at[wid], idx_v)
        copies = []
        for j in range(NCH):
            copies.append(
                pltpu.async_copy(
                    table_hbm.at[idx_v.at[j]],
                    rows_v.at[pl.ds(j * CH, CH)],
                    sem,
                )
            )
        for cp in copies:
            cp.wait()
        pltpu.sync_copy(rows_v, out_hbm.at[pl.ds(wid * EPW, EPW)])

    return gather_kernel(atom_feat, dst_idx)


def _dense_body(b_ref, n_ref, w_ref, o_ref):
    bond_t = b_ref[...].T
    nb_t = n_ref[...].T
    parts = [bond_t[k:k + 1, :] * nb_t for k in range(BD)]
    parts.append(nb_t)
    z_t = jnp.concatenate(parts, axis=0)
    msg_t = jnp.dot(w_ref[...], z_t, preferred_element_type=jnp.float32)
    o_ref[...] = msg_t.T


def _dense(bond_feat, neighbor, w2_aug_t):
    return pl.pallas_call(
        _dense_body,
        grid=(E // BE,),
        in_specs=[
            pl.BlockSpec((BE, BD), lambda i: (i, 0)),
            pl.BlockSpec((BE, AD), lambda i: (i, 0)),
            pl.BlockSpec((AD, K_AUG), lambda i: (0, 0)),
        ],
        out_specs=pl.BlockSpec((BE, AD), lambda i: (i, 0)),
        out_shape=jax.ShapeDtypeStruct((E, AD), jnp.float32),
    )(bond_feat, neighbor, w2_aug_t)


def _scatter(msg, src_idx, zeros):
    @functools.partial(
        pl.kernel,
        out_type=jax.ShapeDtypeStruct((NC, NN, AD), jnp.float32),
        mesh=_sc_mesh(),
        scratch_types=[
            pltpu.VMEM((NCH, CH), jnp.int32),
            pltpu.VMEM((EPW, AD), jnp.float32),
            pltpu.VMEM_SHARED((NN, AD), jnp.float32),
            pltpu.SemaphoreType.DMA,
        ],
        compiler_params=_SC_PARAMS,
    )
    def scatter_kernel(msg_hbm, idx_hbm, zero_hbm, out_hbm, idx_v, msg_v, acc, sem):
        c = lax.axis_index("c")
        s = lax.axis_index("s")
        wid = s * NC + c
        pltpu.sync_copy(
            zero_hbm.at[pl.ds(s * RPT, RPT)], acc.at[pl.ds(s * RPT, RPT)]
        )
        pltpu.sync_copy(idx_hbm.at[wid], idx_v)
        pltpu.sync_copy(msg_hbm.at[pl.ds(wid * EPW, EPW)], msg_v)
        plsc.subcore_barrier()
        for j in range(NCH):
            pltpu.sync_copy(
                msg_v.at[pl.ds(j * CH, CH)], acc.at[idx_v.at[j]], add=True
            )
        plsc.subcore_barrier()
        pltpu.sync_copy(
            acc.at[pl.ds(s * RPT, RPT)], out_hbm.at[c, pl.ds(s * RPT, RPT)]
        )

    return scatter_kernel(msg, src_idx, zeros)


def _combine_body(p_ref, o_ref):
    o_ref[...] = p_ref[0] + p_ref[1]


def _combine(partials):
    return pl.pallas_call(
        _combine_body,
        out_shape=jax.ShapeDtypeStruct((NN, AD), jnp.float32),
    )(partials)


def kernel(atom_feat, bond_feat, pair_idx, kernel, bias):
    w2 = kernel.reshape(BD, AD, AD).transpose(0, 2, 1).reshape(BD * AD, AD)
    w2_aug = jnp.concatenate([w2, bias.reshape(AD, AD).T], axis=0)
    idx = pair_idx.astype(jnp.int32)
    src = idx[:, 0].reshape(NW, NCH, CH)
    dst = idx[:, 1].reshape(NW, NCH, CH)

    neighbor = _gather(atom_feat, dst)
    msg = _dense(bond_feat, neighbor, w2_aug.T)
    partials = _scatter(msg, src, jnp.zeros((NN, AD), jnp.float32))
    return _combine(partials)


# speedup vs baseline: 7.3482x; 1.3495x over previous
import functools

import jax
import jax.numpy as jnp
from jax import lax
from jax.experimental import pallas as pl
from jax.experimental.pallas import tpu as pltpu
from jax.experimental.pallas import tpu_sc as plsc

E = 71680
NN = 4480
AD = 32
BD = 16
NC = 2
NS = 16
NW = NC * NS
EPW = E // NW
CH = 112
NCH = EPW // CH
RPT = NN // NS
BE = 2048
K_AUG = BD * AD + AD


def _sc_mesh():
    return plsc.VectorSubcoreMesh(core_axis_name="c", subcore_axis_name="s")


_SC_PARAMS = pltpu.CompilerParams(use_tc_tiling_on_sc=False)


def _gather(atom_feat, dst_idx):
    @functools.partial(
        pl.kernel,
        out_type=jax.ShapeDtypeStruct((E, AD), jnp.float32),
        mesh=_sc_mesh(),
        scratch_types=[
            pltpu.VMEM((NCH, CH), jnp.int32),
            pltpu.VMEM((EPW, AD), jnp.float32),
            pltpu.SemaphoreType.DMA,
        ],
        compiler_params=_SC_PARAMS,
    )
    def gather_kernel(table_hbm, idx_hbm, out_hbm, idx_v, rows_v, sem):
        c = lax.axis_index("c")
        s = lax.axis_index("s")
        wid = s * NC + c
        pltpu.sync_copy(idx_hbm.at[wid], idx_v)
        copies = []
        for j in range(NCH):
            copies.append(
                pltpu.async_copy(
                    table_hbm.at[idx_v.at[j]],
                    rows_v.at[pl.ds(j * CH, CH)],
                    sem,
                )
            )
        for cp in copies:
            cp.wait()
        pltpu.sync_copy(rows_v, out_hbm.at[pl.ds(wid * EPW, EPW)])

    return gather_kernel(atom_feat, dst_idx)


def _dense_body(bg_ref, n_ref, w_ref, o_ref):
    nb_t = n_ref[...].T
    w = w_ref[...]
    outs = []
    for q in range(4):
        nb_q = nb_t[AD * q:AD * (q + 1), :]
        b_q = bg_ref[q]
        parts = [b_q[k:k + 1, :] * nb_q for k in range(BD)]
        parts.append(nb_q)
        z_q = jnp.concatenate(parts, axis=0)
        outs.append(jnp.dot(w, z_q, preferred_element_type=jnp.float32))
    o_ref[...] = jnp.concatenate(outs, axis=0).T


def _dense(bond_g, nb128, w2_aug_t):
    return pl.pallas_call(
        _dense_body,
        grid=(E // BE,),
        in_specs=[
            pl.BlockSpec((4, BD, BE // 4), lambda i: (0, 0, i)),
            pl.BlockSpec((BE // 4, 128), lambda i: (i, 0)),
            pl.BlockSpec((AD, K_AUG), lambda i: (0, 0)),
        ],
        out_specs=pl.BlockSpec((BE // 4, 128), lambda i: (i, 0)),
        out_shape=jax.ShapeDtypeStruct((E // 4, 128), jnp.float32),
    )(bond_g, nb128, w2_aug_t)


def _scatter(msg, src_idx, zeros):
    @functools.partial(
        pl.kernel,
        out_type=jax.ShapeDtypeStruct((NC, NN, AD), jnp.float32),
        mesh=_sc_mesh(),
        scratch_types=[
            pltpu.VMEM((NCH, CH), jnp.int32),
            pltpu.VMEM((EPW, AD), jnp.float32),
            pltpu.VMEM_SHARED((NN, AD), jnp.float32),
            pltpu.SemaphoreType.DMA,
        ],
        compiler_params=_SC_PARAMS,
    )
    def scatter_kernel(msg_hbm, idx_hbm, zero_hbm, out_hbm, idx_v, msg_v, acc, sem):
        c = lax.axis_index("c")
        s = lax.axis_index("s")
        wid = s * NC + c
        pltpu.sync_copy(
            zero_hbm.at[pl.ds(s * RPT, RPT)], acc.at[pl.ds(s * RPT, RPT)]
        )
        pltpu.sync_copy(idx_hbm.at[wid], idx_v)
        pltpu.sync_copy(msg_hbm.at[pl.ds(wid * EPW, EPW)], msg_v)
        plsc.subcore_barrier()
        for j in range(NCH):
            pltpu.sync_copy(
                msg_v.at[pl.ds(j * CH, CH)], acc.at[idx_v.at[j]], add=True
            )
        plsc.subcore_barrier()
        pltpu.sync_copy(
            acc.at[pl.ds(s * RPT, RPT)], out_hbm.at[c, pl.ds(s * RPT, RPT)]
        )

    return scatter_kernel(msg, src_idx, zeros)


def _combine_body(p_ref, o_ref):
    o_ref[...] = p_ref[0] + p_ref[1]


def _combine(partials):
    return pl.pallas_call(
        _combine_body,
        out_shape=jax.ShapeDtypeStruct((NN, AD), jnp.float32),
    )(partials)


def kernel(atom_feat, bond_feat, pair_idx, kernel, bias):
    w2 = kernel.reshape(BD, AD, AD).transpose(0, 2, 1).reshape(BD * AD, AD)
    w2_aug = jnp.concatenate([w2, bias.reshape(AD, AD).T], axis=0)
    idx = pair_idx.astype(jnp.int32)
    src = idx[:, 0].reshape(NW, NCH, CH)
    dst = idx[:, 1].reshape(NW, NCH, CH)

    bond_g = bond_feat.reshape(E // 4, 4, BD).transpose(1, 2, 0)
    neighbor = _gather(atom_feat, dst)
    msg128 = _dense(bond_g, neighbor.reshape(E // 4, 128), w2_aug.T)
    partials = _scatter(
        msg128.reshape(E, AD), src, jnp.zeros((NN, AD), jnp.float32)
    )
    return _combine(partials)


# speedup vs baseline: 7.6951x; 1.0472x over previous
import functools

import jax
import jax.numpy as jnp
from jax import lax
from jax.experimental import pallas as pl
from jax.experimental.pallas import tpu as pltpu
from jax.experimental.pallas import tpu_sc as plsc

E = 71680
NN = 4480
AD = 32
BD = 16
NC = 2
NS = 16
NW = NC * NS
EPW = E // NW
CH = 112
NCH = EPW // CH
RPT = NN // NS
BE = 2048
K_AUG = BD * AD + AD


def _sc_mesh():
    return plsc.VectorSubcoreMesh(core_axis_name="c", subcore_axis_name="s")


_SC_PARAMS = pltpu.CompilerParams(use_tc_tiling_on_sc=False)


def _gather(atom_feat, dst_idx):
    @functools.partial(
        pl.kernel,
        out_type=jax.ShapeDtypeStruct((E, AD), jnp.float32),
        mesh=_sc_mesh(),
        scratch_types=[
            pltpu.VMEM((NCH, CH), jnp.int32),
            pltpu.VMEM((EPW, AD), jnp.float32),
            pltpu.SemaphoreType.DMA,
        ],
        compiler_params=_SC_PARAMS,
    )
    def gather_kernel(table_hbm, idx_hbm, out_hbm, idx_v, rows_v, sem):
        c = lax.axis_index("c")
        s = lax.axis_index("s")
        wid = s * NC + c
        pltpu.sync_copy(idx_hbm.at[wid], idx_v)
        copies = []
        for j in range(NCH):
            copies.append(
                pltpu.async_copy(
                    table_hbm.at[idx_v.at[j]],
                    rows_v.at[pl.ds(j * CH, CH)],
                    sem,
                )
            )
        for cp in copies:
            cp.wait()
        pltpu.sync_copy(rows_v, out_hbm.at[pl.ds(wid * EPW, EPW)])

    return gather_kernel(atom_feat, dst_idx)


def _dense_body(bg_ref, n_ref, w_ref, o_ref):
    nb_t = n_ref[...].T
    w = w_ref[...]
    outs = []
    for q in range(4):
        nb_q = nb_t[AD * q:AD * (q + 1), :]
        b_q = bg_ref[q]
        parts = [b_q[k:k + 1, :] * nb_q for k in range(BD)]
        parts.append(nb_q)
        z_q = jnp.concatenate(parts, axis=0)
        outs.append(jnp.dot(w, z_q, preferred_element_type=jnp.float32))
    o_ref[...] = jnp.concatenate(outs, axis=0).T


def _dense(bond_g, nb128, w2_aug_t):
    return pl.pallas_call(
        _dense_body,
        grid=(E // BE,),
        in_specs=[
            pl.BlockSpec((4, BD, BE // 4), lambda i: (0, 0, i)),
            pl.BlockSpec((BE // 4, 128), lambda i: (i, 0)),
            pl.BlockSpec((AD, K_AUG), lambda i: (0, 0)),
        ],
        out_specs=pl.BlockSpec((BE // 4, 128), lambda i: (i, 0)),
        out_shape=jax.ShapeDtypeStruct((E // 4, 128), jnp.float32),
    )(bond_g, nb128, w2_aug_t)


def _scatter(msg, src_idx, zeros):
    @functools.partial(
        pl.kernel,
        out_type=jax.ShapeDtypeStruct((NC, NN, AD), jnp.float32),
        mesh=_sc_mesh(),
        scratch_types=[
            pltpu.VMEM((NCH, CH), jnp.int32),
            pltpu.VMEM((EPW, AD), jnp.float32),
            pltpu.VMEM_SHARED((NN, AD), jnp.float32),
            pltpu.SemaphoreType.DMA,
        ],
        compiler_params=_SC_PARAMS,
    )
    def scatter_kernel(msg_hbm, idx_hbm, zero_hbm, out_hbm, idx_v, msg_v, acc, sem):
        c = lax.axis_index("c")
        s = lax.axis_index("s")
        wid = s * NC + c
        pltpu.sync_copy(
            zero_hbm.at[pl.ds(s * RPT, RPT)], acc.at[pl.ds(s * RPT, RPT)]
        )
        pltpu.sync_copy(idx_hbm.at[wid], idx_v)
        pltpu.sync_copy(msg_hbm.at[pl.ds(wid * EPW, EPW)], msg_v)
        plsc.subcore_barrier()
        for j in range(NCH):
            pltpu.sync_copy(
                msg_v.at[pl.ds(j * CH, CH)], acc.at[idx_v.at[j]], add=True
            )
        plsc.subcore_barrier()
        pltpu.sync_copy(
            acc.at[pl.ds(s * RPT, RPT)], out_hbm.at[c, pl.ds(s * RPT, RPT)]
        )

    return scatter_kernel(msg, src_idx, zeros)


def _combine_body(p_ref, o_ref):
    o_ref[...] = p_ref[0] + p_ref[1]


def _combine(partials):
    out128 = pl.pallas_call(
        _combine_body,
        out_shape=jax.ShapeDtypeStruct((NN * AD // 128, 128), jnp.float32),
    )(partials.reshape(NC, NN * AD // 128, 128))
    return out128.reshape(NN, AD)


def kernel(atom_feat, bond_feat, pair_idx, kernel, bias):
    w2 = kernel.reshape(BD, AD, AD).transpose(0, 2, 1).reshape(BD * AD, AD)
    w2_aug = jnp.concatenate([w2, bias.reshape(AD, AD).T], axis=0)
    idx = pair_idx.astype(jnp.int32)
    src = idx[:, 0].reshape(NW, NCH, CH)
    dst = idx[:, 1].reshape(NW, NCH, CH)

    bond_g = bond_feat.reshape(E // 4, 4, BD).transpose(1, 2, 0)
    neighbor = _gather(atom_feat, dst)
    msg128 = _dense(bond_g, neighbor.reshape(E // 4, 128), w2_aug.T)
    partials = _scatter(
        msg128.reshape(E, AD), src, jnp.zeros((NN, AD), jnp.float32)
    )
    return _combine(partials)
